# baseline (speedup 1.0000x reference)
import math

import numpy as np

import jax
import jax.numpy as jnp
from jax.experimental import pallas as pl
from jax.experimental.pallas import tpu as pltpu


def _interp_arrays(out_size, in_size):
    if out_size == 1:
        src = np.zeros((1,), np.float32)
    else:
        src = np.arange(out_size, dtype=np.float32) * np.float32(
            (in_size - 1) / (out_size - 1)
        )
    i0 = np.clip(np.floor(src).astype(np.int32), 0, in_size - 1)
    i1 = np.minimum(i0 + 1, in_size - 1)
    w1 = src - i0.astype(np.float32)
    w0 = np.float32(1.0) - w1
    return i0, i1, w0, w1


def _interp_matrix(out_size, in_size):
    i0, i1, w0, w1 = _interp_arrays(out_size, in_size)
    m = np.zeros((out_size, in_size), np.float32)
    m[np.arange(out_size), i0] += w0
    m[np.arange(out_size), i1] += w1
    return m


def _resize_kernel(*refs):
    x_refs, wwt_ref, wh_ref, o_ref = refs[:-3], refs[-3], refs[-2], refs[-1]
    wwt = wwt_ref[...].astype(jnp.bfloat16)
    wh = wh_ref[...].astype(jnp.bfloat16)
    wo = wwt.shape[1]
    for s, x_ref in enumerate(x_refs):
        x = x_ref[...].astype(jnp.bfloat16)
        tb, h, w = x.shape
        u = jnp.dot(x.reshape(tb * h, w), wwt,
                    preferred_element_type=jnp.float32).reshape(tb, h, wo)
        for b in range(tb):
            o_ref[s * tb + b] = jnp.dot(wh, u[b].astype(jnp.bfloat16),
                                        preferred_element_type=jnp.float32)


def kernel(x):
    vel_resize = 2.0
    factor = 1.0 / vel_resize
    N, C, H, W = x.shape
    H_out = int(math.floor(H * factor))
    W_out = int(math.floor(W * factor))
    B = N * C

    wwt = jnp.asarray(np.ascontiguousarray(_interp_matrix(W_out, W).T))
    wh = jnp.asarray(np.float32(factor) * _interp_matrix(H_out, H))

    S, TB = 4, 8
    while TB > 1 and B % (S * TB):
        TB //= 2
    if B % (S * TB):
        S, TB = 1, 1
    grid_b = B // (S * TB)

    xf = x.reshape(B, H, W)
    in_specs = [
        pl.BlockSpec((TB, H, W), lambda b, s=s: (S * b + s, 0, 0))
        for s in range(S)
    ] + [
        pl.BlockSpec((W, W_out), lambda b: (0, 0)),
        pl.BlockSpec((H_out, H), lambda b: (0, 0)),
    ]
    out3 = pl.pallas_call(
        _resize_kernel,
        out_shape=jax.ShapeDtypeStruct((B, H_out, W_out), x.dtype),
        grid=(grid_b,),
        in_specs=in_specs,
        out_specs=pl.BlockSpec((S * TB, H_out, W_out), lambda b: (b, 0, 0)),
        compiler_params=pltpu.CompilerParams(
            dimension_semantics=("parallel",),
            vmem_limit_bytes=int(64 * 1024 * 1024 * 0.85),
        ),
    )(*([xf] * S), wwt, wh)
    return out3.reshape(N, C, H_out, W_out)


# speedup vs baseline: 1.0302x; 1.0302x over previous
import math

import numpy as np

import jax
import jax.numpy as jnp
from jax.experimental import pallas as pl
from jax.experimental.pallas import tpu as pltpu


def _interp_arrays(out_size, in_size):
    if out_size == 1:
        src = np.zeros((1,), np.float32)
    else:
        src = np.arange(out_size, dtype=np.float32) * np.float32(
            (in_size - 1) / (out_size - 1)
        )
    i0 = np.clip(np.floor(src).astype(np.int32), 0, in_size - 1)
    i1 = np.minimum(i0 + 1, in_size - 1)
    w1 = src - i0.astype(np.float32)
    w0 = np.float32(1.0) - w1
    return i0, i1, w0, w1


def _interp_matrix(out_size, in_size):
    i0, i1, w0, w1 = _interp_arrays(out_size, in_size)
    m = np.zeros((out_size, in_size), np.float32)
    m[np.arange(out_size), i0] += w0
    m[np.arange(out_size), i1] += w1
    return m


def _resize_kernel(*refs):
    x_refs, wwt_ref, wh_ref, o_ref = refs[:-3], refs[-3], refs[-2], refs[-1]
    wwt = wwt_ref[...].astype(jnp.bfloat16)
    wh = wh_ref[...].astype(jnp.bfloat16)
    wo = wwt.shape[1]
    for s, x_ref in enumerate(x_refs):
        x = x_ref[...].astype(jnp.bfloat16)
        tb, h, w = x.shape
        u = jnp.dot(x.reshape(tb * h, w), wwt,
                    preferred_element_type=jnp.float32).reshape(tb, h, wo)
        for b in range(tb):
            o_ref[s * tb + b] = jnp.dot(wh, u[b].astype(jnp.bfloat16),
                                        preferred_element_type=jnp.float32)


def kernel(x):
    vel_resize = 2.0
    factor = 1.0 / vel_resize
    N, C, H, W = x.shape
    H_out = int(math.floor(H * factor))
    W_out = int(math.floor(W * factor))
    B = N * C

    wwt = jnp.asarray(np.ascontiguousarray(_interp_matrix(W_out, W).T))
    wh = jnp.asarray(np.float32(factor) * _interp_matrix(H_out, H))

    S, TB = 2, 32
    while TB > 1 and B % (S * TB):
        TB //= 2
    if B % (S * TB):
        S, TB = 1, 1
    grid_b = B // (S * TB)

    xf = x.reshape(B, H, W)
    in_specs = [
        pl.BlockSpec((TB, H, W), lambda b, s=s: (S * b + s, 0, 0))
        for s in range(S)
    ] + [
        pl.BlockSpec((W, W_out), lambda b: (0, 0)),
        pl.BlockSpec((H_out, H), lambda b: (0, 0)),
    ]
    out3 = pl.pallas_call(
        _resize_kernel,
        out_shape=jax.ShapeDtypeStruct((B, H_out, W_out), x.dtype),
        grid=(grid_b,),
        in_specs=in_specs,
        out_specs=pl.BlockSpec((S * TB, H_out, W_out), lambda b: (b, 0, 0)),
        compiler_params=pltpu.CompilerParams(
            dimension_semantics=("parallel",),
            vmem_limit_bytes=int(64 * 1024 * 1024 * 0.85),
        ),
    )(*([xf] * S), wwt, wh)
    return out3.reshape(N, C, H_out, W_out)


# speedup vs baseline: 1.0982x; 1.0660x over previous
import math

import numpy as np

import jax
import jax.numpy as jnp
from jax.experimental import pallas as pl
from jax.experimental.pallas import tpu as pltpu


def _interp_arrays(out_size, in_size):
    if out_size == 1:
        src = np.zeros((1,), np.float32)
    else:
        src = np.arange(out_size, dtype=np.float32) * np.float32(
            (in_size - 1) / (out_size - 1)
        )
    i0 = np.clip(np.floor(src).astype(np.int32), 0, in_size - 1)
    i1 = np.minimum(i0 + 1, in_size - 1)
    w1 = src - i0.astype(np.float32)
    w0 = np.float32(1.0) - w1
    return i0, i1, w0, w1


def _interp_matrix(out_size, in_size):
    i0, i1, w0, w1 = _interp_arrays(out_size, in_size)
    m = np.zeros((out_size, in_size), np.float32)
    m[np.arange(out_size), i0] += w0
    m[np.arange(out_size), i1] += w1
    return m


def _resize_kernel(*refs):
    x_refs, wwt_ref, wh_ref, o_ref = refs[:-3], refs[-3], refs[-2], refs[-1]
    wwt = wwt_ref[...].astype(jnp.bfloat16)
    wh = wh_ref[...].astype(jnp.bfloat16)
    wo = wwt.shape[1]
    for s, x_ref in enumerate(x_refs):
        x = x_ref[...].astype(jnp.bfloat16)
        tb, h, w = x.shape
        u = jnp.dot(x.reshape(tb * h, w), wwt,
                    preferred_element_type=jnp.float32).reshape(tb, h, wo)
        for b in range(tb):
            o_ref[s * tb + b] = jnp.dot(wh, u[b].astype(jnp.bfloat16),
                                        preferred_element_type=jnp.float32)


def kernel(x):
    vel_resize = 2.0
    factor = 1.0 / vel_resize
    N, C, H, W = x.shape
    H_out = int(math.floor(H * factor))
    W_out = int(math.floor(W * factor))
    B = N * C

    wwt = jnp.asarray(np.ascontiguousarray(_interp_matrix(W_out, W).T))
    wh = jnp.asarray(np.float32(factor) * _interp_matrix(H_out, H))

    S, TB = 1, 32
    while TB > 1 and B % (S * TB):
        TB //= 2
    if B % (S * TB):
        S, TB = 1, 1
    grid_b = B // (S * TB)

    xf = x.reshape(B, H, W)
    in_specs = [
        pl.BlockSpec((TB, H, W), lambda b, s=s: (S * b + s, 0, 0))
        for s in range(S)
    ] + [
        pl.BlockSpec((W, W_out), lambda b: (0, 0)),
        pl.BlockSpec((H_out, H), lambda b: (0, 0)),
    ]
    out3 = pl.pallas_call(
        _resize_kernel,
        out_shape=jax.ShapeDtypeStruct((B, H_out, W_out), x.dtype),
        grid=(grid_b,),
        in_specs=in_specs,
        out_specs=pl.BlockSpec((S * TB, H_out, W_out), lambda b: (b, 0, 0)),
        compiler_params=pltpu.CompilerParams(
            dimension_semantics=("parallel",),
            vmem_limit_bytes=int(64 * 1024 * 1024 * 0.85),
        ),
    )(*([xf] * S), wwt, wh)
    return out3.reshape(N, C, H_out, W_out)
